# baseline (speedup 1.0000x reference)
import functools

import jax
import jax.numpy as jnp
from jax import lax
from jax.experimental import pallas as pl
from jax.experimental.pallas import tpu as pltpu
from jax.experimental.pallas import tpu_sc as plsc

VOCAB = 1000000
EMBED_DIM = 64
BATCH = 4096
HIST = 200

NC = 2
NS = 16
NW = NC * NS

TOTAL = BATCH * HIST
PER_W = TOTAL // NW
GROUP = 128
GROUPS = PER_W // GROUP
NBUF = 8
LEAD = 4


def _make_gather():
    mesh = plsc.VectorSubcoreMesh(core_axis_name="c", subcore_axis_name="s")

    @functools.partial(
        pl.kernel,
        out_type=jax.ShapeDtypeStruct((TOTAL, 128), jnp.float32),
        mesh=mesh,
        scratch_types=[
            pltpu.VMEM((GROUPS, GROUP), jnp.int32),
            pltpu.VMEM((NBUF, GROUP, EMBED_DIM), jnp.float32),
            pltpu.SemaphoreType.DMA((NBUF,)),
            pltpu.SemaphoreType.DMA((NBUF,)),
        ],
        compiler_params=pltpu.CompilerParams(use_tc_tiling_on_sc=False),
    )
    def gather_kernel(idx_hbm, table_hbm, out_hbm, idx_v, rows_v, gsem, ssem):
        wid = lax.axis_index("s") * NC + lax.axis_index("c")
        base = wid * PER_W
        pltpu.sync_copy(idx_hbm.at[wid], idx_v)

        def start_gather(g, b):
            pltpu.async_copy(table_hbm.at[idx_v.at[g]], rows_v.at[b],
                             gsem.at[b])

        def wait_gather(b):
            pltpu.make_async_copy(table_hbm.at[pl.ds(0, GROUP)],
                                  rows_v.at[b], gsem.at[b]).wait()

        def start_store(g, b):
            pltpu.async_copy(rows_v.at[b],
                             out_hbm.at[pl.ds(base + g * GROUP, GROUP),
                                        pl.ds(0, EMBED_DIM)],
                             ssem.at[b])

        def wait_store(b):
            pltpu.make_async_copy(rows_v.at[b],
                                  out_hbm.at[pl.ds(base, GROUP),
                                             pl.ds(0, EMBED_DIM)],
                                  ssem.at[b]).wait()

        for g in range(LEAD):
            start_gather(g, g % NBUF)

        def outer(i, _):
            t0 = i * NBUF
            for j in range(NBUF):
                t = t0 + j
                u = t + LEAD
                bu = (j + LEAD) % NBUF

                @pl.when(u < GROUPS)
                def _():
                    @pl.when(u >= NBUF)
                    def _():
                        wait_store(bu)
                    start_gather(u, bu)

                wait_gather(j)
                start_store(t, j)
            return ()

        lax.fori_loop(0, GROUPS // NBUF, outer, (), unroll=False)

        for b in range(NBUF):
            wait_store(b)

    return gather_kernel


_gather = _make_gather()


def kernel(batch, table):
    idx = batch.astype(jnp.int32).reshape(NW, GROUPS, GROUP)
    flat = _gather(idx, table)
    return flat[:, :EMBED_DIM].reshape(BATCH, HIST, EMBED_DIM)


# speedup vs baseline: 1.0035x; 1.0035x over previous
import functools

import jax
import jax.numpy as jnp
from jax import lax
from jax.experimental import pallas as pl
from jax.experimental.pallas import tpu as pltpu
from jax.experimental.pallas import tpu_sc as plsc

VOCAB = 1000000
EMBED_DIM = 64
BATCH = 4096
HIST = 200

NC = 2
NS = 16
NW = NC * NS

TOTAL = BATCH * HIST
PER_W = TOTAL // NW
GROUP = 128
GROUPS = PER_W // GROUP
NBUF = 10
LEAD = 5


def _make_gather():
    mesh = plsc.VectorSubcoreMesh(core_axis_name="c", subcore_axis_name="s")

    @functools.partial(
        pl.kernel,
        out_type=jax.ShapeDtypeStruct((TOTAL, 128), jnp.float32),
        mesh=mesh,
        scratch_types=[
            pltpu.VMEM((GROUPS, GROUP), jnp.int32),
            pltpu.VMEM((NBUF, GROUP, EMBED_DIM), jnp.float32),
            pltpu.SemaphoreType.DMA((NBUF,)),
            pltpu.SemaphoreType.DMA((NBUF,)),
        ],
        compiler_params=pltpu.CompilerParams(use_tc_tiling_on_sc=False),
    )
    def gather_kernel(idx_hbm, table_hbm, out_hbm, idx_v, rows_v, gsem, ssem):
        wid = lax.axis_index("s") * NC + lax.axis_index("c")
        base = wid * PER_W
        pltpu.sync_copy(idx_hbm.at[wid], idx_v)

        def start_gather(g, b):
            pltpu.async_copy(table_hbm.at[idx_v.at[g]], rows_v.at[b],
                             gsem.at[b])

        def wait_gather(b):
            pltpu.make_async_copy(table_hbm.at[pl.ds(0, GROUP)],
                                  rows_v.at[b], gsem.at[b]).wait()

        def start_store(g, b):
            pltpu.async_copy(rows_v.at[b],
                             out_hbm.at[pl.ds(base + g * GROUP, GROUP),
                                        pl.ds(0, EMBED_DIM)],
                             ssem.at[b])

        def wait_store(b):
            pltpu.make_async_copy(rows_v.at[b],
                                  out_hbm.at[pl.ds(base, GROUP),
                                             pl.ds(0, EMBED_DIM)],
                                  ssem.at[b]).wait()

        for g in range(LEAD):
            start_gather(g, g % NBUF)

        def outer(i, _):
            t0 = i * NBUF
            for j in range(NBUF):
                t = t0 + j
                u = t + LEAD
                bu = (j + LEAD) % NBUF

                @pl.when(u < GROUPS)
                def _():
                    @pl.when(u >= NBUF)
                    def _():
                        wait_store(bu)
                    start_gather(u, bu)

                wait_gather(j)
                start_store(t, j)
            return ()

        lax.fori_loop(0, GROUPS // NBUF, outer, (), unroll=False)

        for b in range(NBUF):
            wait_store(b)

    return gather_kernel


_gather = _make_gather()


def kernel(batch, table):
    idx = batch.astype(jnp.int32).reshape(NW, GROUPS, GROUP)
    flat = _gather(idx, table)
    return flat[:, :EMBED_DIM].reshape(BATCH, HIST, EMBED_DIM)
